# Initial kernel scaffold; baseline (speedup 1.0000x reference)
import jax
import jax.numpy as jnp
from jax.experimental import pallas as pl


def kernel(x, edge_index, W1, a1_src, a1_dst, b1, W2, a2_src, a2_dst, b2, W3, a3_src, a3_dst, b3, W4, a4_src, a4_dst, b4):
    raise NotImplementedError("write your pallas kernel here")


# speedup vs baseline: 14.4661x; 14.4661x over previous
import functools

import jax
import jax.numpy as jnp
from jax import lax
from jax.experimental import pallas as pl
from jax.experimental.pallas import tpu as pltpu
from jax.experimental.pallas import tpu_sc as plsc

N = 10000
N1 = 10240
E_RAW = 320000
E_REAL = E_RAW + N
CH = 96
NW = 32
NCH = 108
EPW = NCH * CH
E1 = EPW * NW
BR = 512
NBLK = N1 // BR
RPS = N1 // 16

f32 = jnp.float32
i32 = jnp.int32


def _full(shape):
    return pl.BlockSpec(shape, lambda i: tuple(0 for _ in shape))


def _rows(shape):
    return pl.BlockSpec(shape, lambda i: (i,) + tuple(0 for _ in shape[1:]))


def _prep_pre(x, W, a_s, a_d):
    din, dout = W.shape

    def body(x_ref, w_ref, as_ref, ad_ref, m_ref, uv_ref):
        xb = x_ref[...]
        w = w_ref[...]
        wu = jnp.dot(w, as_ref[...], preferred_element_type=f32)
        wv = jnp.dot(w, ad_ref[...], preferred_element_type=f32)
        u = jnp.dot(xb, wu, preferred_element_type=f32)
        v = jnp.dot(xb, wv, preferred_element_type=f32)
        ones = jnp.ones((BR, 1), f32)
        zeros = jnp.zeros((BR, 15), f32)
        m_ref[...] = jnp.concatenate([xb, ones, zeros], axis=1)
        uv_ref[...] = jnp.concatenate([u, v], axis=1).T

    return pl.pallas_call(
        body,
        grid=(NBLK,),
        in_specs=[_rows((BR, din)), _full((din, dout)),
                  _full((dout, 1)), _full((dout, 1))],
        out_specs=[_rows((BR, 144)),
                   pl.BlockSpec((2, BR), lambda i: (0, i))],
        out_shape=[jax.ShapeDtypeStruct((N1, 144), f32),
                   jax.ShapeDtypeStruct((2, N1), f32)],
    )(x, W, a_s, a_d)


def _prep_post(x, W, a_s, a_d, widths):
    din, dout = W.shape

    def body(x_ref, w_ref, as_ref, ad_ref, *out_refs):
        uv_ref = out_refs[-1]
        m_refs = out_refs[:-1]
        h = jnp.dot(x_ref[...], w_ref[...], preferred_element_type=f32)
        u = jnp.dot(h, as_ref[...], preferred_element_type=f32)
        v = jnp.dot(h, ad_ref[...], preferred_element_type=f32)
        col = 0
        for k, w_k in enumerate(widths):
            dm = w_k if k > 0 else w_k - 16
            piece = h[:, col:col + dm]
            col += dm
            if k == 0:
                piece = jnp.concatenate(
                    [piece, jnp.ones((BR, 1), f32), jnp.zeros((BR, 15), f32)],
                    axis=1)
            m_refs[k][...] = piece
        uv_ref[...] = jnp.concatenate([u, v], axis=1).T

    return pl.pallas_call(
        body,
        grid=(NBLK,),
        in_specs=[_rows((BR, din)), _full((din, dout)),
                  _full((dout, 1)), _full((dout, 1))],
        out_specs=[_rows((BR, w)) for w in widths]
                  + [pl.BlockSpec((2, BR), lambda i: (0, i))],
        out_shape=[jax.ShapeDtypeStruct((N1, w), f32) for w in widths]
                  + [jax.ShapeDtypeStruct((2, N1), f32)],
    )(x, W, a_s, a_d)


def _finish_matmul(accp, W, b, relu):
    din, dout = W.shape

    def body(a_ref, w_ref, b_ref, o_ref):
        s = a_ref[0] + a_ref[1]
        den = jnp.maximum(s[:, 128:129], 1e-30)
        g = s[:, :din] / den
        o = jnp.dot(g, w_ref[...], preferred_element_type=f32) + b_ref[...]
        o_ref[...] = jnp.maximum(o, 0.0) if relu else o

    return pl.pallas_call(
        body,
        grid=(NBLK,),
        in_specs=[pl.BlockSpec((2, BR, 144), lambda i: (0, i, 0)),
                  _full((din, dout)), _full((1, dout))],
        out_specs=_rows((BR, dout)),
        out_shape=jax.ShapeDtypeStruct((N1, dout), f32),
    )(accp, W, b)


def _finish_add(accps, b, widths, dout, relu, logsm):
    def body(*refs):
        a_refs = refs[:len(widths)]
        b_ref, o_ref = refs[-2], refs[-1]
        s0 = a_refs[0][0] + a_refs[0][1]
        den = jnp.maximum(s0[:, widths[0] - 16:widths[0] - 15], 1e-30)
        pieces = [s0[:, :widths[0] - 16]]
        for k in range(1, len(widths)):
            pieces.append(a_refs[k][0] + a_refs[k][1])
        g = jnp.concatenate(pieces, axis=1) if len(pieces) > 1 else pieces[0]
        t = g / den + b_ref[...]
        if relu:
            t = jnp.maximum(t, 0.0)
        if logsm:
            m = jnp.max(t, axis=1, keepdims=True)
            t = t - (m + jnp.log(jnp.sum(jnp.exp(t - m), axis=1,
                                         keepdims=True)))
        o_ref[...] = t

    return pl.pallas_call(
        body,
        grid=(NBLK,),
        in_specs=[pl.BlockSpec((2, BR, w), lambda i: (0, i, 0))
                  for w in widths] + [_full((1, dout))],
        out_specs=_rows((BR, dout)),
        out_shape=jax.ShapeDtypeStruct((N1, dout), f32),
    )(*accps, b)


@functools.lru_cache(maxsize=None)
def _make_sc_edge_pass(d_pad):
    mesh = plsc.VectorSubcoreMesh(core_axis_name="c", subcore_axis_name="s")

    @functools.partial(
        pl.kernel,
        out_type=jax.ShapeDtypeStruct((2, N1, d_pad), f32),
        mesh=mesh,
        compiler_params=pltpu.CompilerParams(needs_layout_passes=False,
                                             use_tc_tiling_on_sc=False),
        scratch_types=[
            pltpu.VMEM((N1,), f32),
            pltpu.VMEM((N1,), f32),
            pltpu.VMEM((CH,), i32),
            pltpu.VMEM((CH,), i32),
            pltpu.VMEM((CH,), f32),
            pltpu.VMEM((CH, d_pad), f32),
            pltpu.VMEM((16, d_pad), f32),
            pltpu.VMEM_SHARED((N1, d_pad), f32),
            pltpu.SemaphoreType.DMA,
        ],
    )
    def sc_pass(m_hbm, src_hbm, dst_hbm, uv_hbm, out_hbm,
                u_v, v_v, sv, dv, pv, rows, zbuf, acc, sem):
        cid = lax.axis_index("c")
        sid = lax.axis_index("s")
        nv = d_pad // 16

        def zrow(e, carry):
            for k in range(nv):
                zbuf[e, pl.ds(16 * k, 16)] = jnp.zeros((16,), f32)
            return carry
        lax.fori_loop(0, 16, zrow, 0)
        row0 = sid * RPS

        def zacc(k, carry):
            pltpu.sync_copy(zbuf, acc.at[pl.ds(row0 + 16 * k, 16)])
            return carry
        lax.fori_loop(0, RPS // 16, zacc, 0)

        pltpu.sync_copy(uv_hbm.at[0], u_v)
        pltpu.sync_copy(uv_hbm.at[1], v_v)
        lanes = lax.iota(i32, 16)

        def mx(i, m):
            ug = plsc.load_gather(u_v, [16 * i + lanes])
            return jnp.maximum(m, ug)
        mvec = lax.fori_loop(0, N1 // 16, mx, jnp.full((16,), -1e30, f32))
        gmax = jnp.max(mvec)

        plsc.subcore_barrier()

        ebase = (cid * 16 + sid) * EPW

        def chunk_body(ch, carry):
            b = ebase + ch * CH
            pltpu.sync_copy(src_hbm.at[pl.ds(b, CH)], sv)
            pltpu.sync_copy(dst_hbm.at[pl.ds(b, CH)], dv)
            pltpu.async_copy(m_hbm.at[sv], rows, sem).wait()

            def pgrp(j, c2):
                si = sv[pl.ds(16 * j, 16)]
                di = dv[pl.ds(16 * j, 16)]
                ug = plsc.load_gather(u_v, [si])
                vg = plsc.load_gather(v_v, [di])
                zz = ug + vg
                e = jnp.maximum(zz, 0.2 * zz)
                zub = gmax + vg
                cc = jnp.maximum(zub, 0.2 * zub)
                pv[pl.ds(16 * j, 16)] = jnp.exp(e - cc)
                return c2
            lax.fori_loop(0, CH // 16, pgrp, 0)

            def scale(e_i, c2):
                pb = plsc.load_gather(pv, [jnp.full((16,), e_i, i32)])
                for k in range(nv):
                    sl = pl.ds(16 * k, 16)
                    rows[e_i, sl] = rows[e_i, sl] * pb
                return c2
            lax.fori_loop(0, CH, scale, 0)

            pltpu.sync_copy(rows, acc.at[dv], add=True)
            return carry
        lax.fori_loop(0, NCH, chunk_body, 0)

        plsc.subcore_barrier()
        for k in range(RPS // 128):
            r0 = row0 + k * 128
            pltpu.sync_copy(acc.at[pl.ds(r0, 128)],
                            out_hbm.at[cid, pl.ds(r0, 128)])

    return sc_pass


def kernel(x, edge_index, W1, a1_src, a1_dst, b1, W2, a2_src, a2_dst, b2,
           W3, a3_src, a3_dst, b3, W4, a4_src, a4_dst, b4):
    loops = jnp.arange(N, dtype=edge_index.dtype)
    pad = jnp.full((E1 - E_REAL,), N, i32)
    src = jnp.concatenate([edge_index[0], loops, pad])
    dst = jnp.concatenate([edge_index[1], loops, pad])

    x0 = jnp.zeros((N1, 128), f32).at[:N].set(x.astype(f32))

    def col(a):
        return a.astype(f32).reshape(-1, 1)

    def row(b):
        return b.astype(f32).reshape(1, -1)

    sc128 = _make_sc_edge_pass(144)
    sc_l4 = _make_sc_edge_pass(32)

    m, uv = _prep_pre(x0, W1, col(a1_src), col(a1_dst))
    accp = sc128(m, src, dst, uv)
    h = _finish_matmul(accp, W1, row(b1), relu=True)

    m, uv = _prep_pre(h, W2, col(a2_src), col(a2_dst))
    accp = sc128(m, src, dst, uv)
    h = _finish_matmul(accp, W2, row(b2), relu=True)

    widths3 = (144, 128, 128, 128)
    outs = _prep_post(h, W3, col(a3_src), col(a3_dst), widths3)
    ms, uv = outs[:-1], outs[-1]
    accps = [_make_sc_edge_pass(w)(mk, src, dst, uv) for w, mk in
             zip(widths3, ms)]
    h = _finish_add(accps, row(b3), widths3, 512, relu=True, logsm=False)

    outs = _prep_post(h, W4, col(a4_src), col(a4_dst), (32,))
    m, uv = outs[0], outs[1]
    accp = sc_l4(m, src, dst, uv)
    out = _finish_add([accp], row(b4), (32,), 16, relu=False, logsm=True)

    return out[:N]


# speedup vs baseline: 20.1849x; 1.3953x over previous
import functools

import jax
import jax.numpy as jnp
from jax import lax
from jax.experimental import pallas as pl
from jax.experimental.pallas import tpu as pltpu
from jax.experimental.pallas import tpu_sc as plsc

N = 10000
N1 = 10240
E_RAW = 320000
E_REAL = E_RAW + N
CH = 96
NW = 32
NCH = 108
EPW = NCH * CH
E1 = EPW * NW
BR = 512
NBLK = N1 // BR
RPS = N1 // 16

f32 = jnp.float32
i32 = jnp.int32


def _full(shape):
    return pl.BlockSpec(shape, lambda i: tuple(0 for _ in shape))


def _rows(shape):
    return pl.BlockSpec(shape, lambda i: (i,) + tuple(0 for _ in shape[1:]))


def _prep_pre(x, W, a_s, a_d):
    din, dout = W.shape

    def body(x_ref, w_ref, as_ref, ad_ref, m_ref, uv_ref, g_ref, sm):
        i = pl.program_id(0)
        xb = x_ref[...]
        w = w_ref[...]
        wu = jnp.dot(w, as_ref[...], preferred_element_type=f32)
        wv = jnp.dot(w, ad_ref[...], preferred_element_type=f32)
        u = jnp.dot(xb, wu, preferred_element_type=f32)
        v = jnp.dot(xb, wv, preferred_element_type=f32)
        ones = jnp.ones((BR, 1), f32)
        zeros = jnp.zeros((BR, 15), f32)
        m_ref[...] = jnp.concatenate([xb, ones, zeros], axis=1)
        uv_ref[...] = jnp.concatenate([u, v], axis=1).T
        bm = jnp.max(u)

        @pl.when(i == 0)
        def _():
            sm[0] = bm

        @pl.when(i > 0)
        def _():
            sm[0] = jnp.maximum(sm[0], bm)
        g_ref[...] = jnp.full((1, 16), sm[0], f32)

    return pl.pallas_call(
        body,
        grid=(NBLK,),
        in_specs=[_rows((BR, din)), _full((din, dout)),
                  _full((dout, 1)), _full((dout, 1))],
        out_specs=[_rows((BR, 144)),
                   pl.BlockSpec((2, BR), lambda i: (0, i)),
                   pl.BlockSpec((1, 16), lambda i: (0, 0))],
        out_shape=[jax.ShapeDtypeStruct((N1, 144), f32),
                   jax.ShapeDtypeStruct((2, N1), f32),
                   jax.ShapeDtypeStruct((1, 16), f32)],
        scratch_shapes=[pltpu.SMEM((1,), f32)],
    )(x, W, a_s, a_d)


def _prep_post(x, W, a_s, a_d, widths):
    din, dout = W.shape

    def body(x_ref, w_ref, as_ref, ad_ref, *refs):
        sm = refs[-1]
        g_ref = refs[-2]
        uv_ref = refs[-3]
        m_refs = refs[:-3]
        i = pl.program_id(0)
        h = jnp.dot(x_ref[...], w_ref[...], preferred_element_type=f32)
        u = jnp.dot(h, as_ref[...], preferred_element_type=f32)
        v = jnp.dot(h, ad_ref[...], preferred_element_type=f32)
        col = 0
        for k, w_k in enumerate(widths):
            dm = w_k if k > 0 else w_k - 16
            piece = h[:, col:col + dm]
            col += dm
            if k == 0:
                piece = jnp.concatenate(
                    [piece, jnp.ones((BR, 1), f32), jnp.zeros((BR, 15), f32)],
                    axis=1)
            m_refs[k][...] = piece
        uv_ref[...] = jnp.concatenate([u, v], axis=1).T
        bm = jnp.max(u)

        @pl.when(i == 0)
        def _():
            sm[0] = bm

        @pl.when(i > 0)
        def _():
            sm[0] = jnp.maximum(sm[0], bm)
        g_ref[...] = jnp.full((1, 16), sm[0], f32)

    return pl.pallas_call(
        body,
        grid=(NBLK,),
        in_specs=[_rows((BR, din)), _full((din, dout)),
                  _full((dout, 1)), _full((dout, 1))],
        out_specs=[_rows((BR, w)) for w in widths]
                  + [pl.BlockSpec((2, BR), lambda i: (0, i)),
                     pl.BlockSpec((1, 16), lambda i: (0, 0))],
        out_shape=[jax.ShapeDtypeStruct((N1, w), f32) for w in widths]
                  + [jax.ShapeDtypeStruct((2, N1), f32),
                     jax.ShapeDtypeStruct((1, 16), f32)],
        scratch_shapes=[pltpu.SMEM((1,), f32)],
    )(x, W, a_s, a_d)


def _finish_matmul(accp, W, b, relu):
    din, dout = W.shape

    def body(a_ref, w_ref, b_ref, o_ref):
        s = a_ref[0] + a_ref[1]
        den = jnp.maximum(s[:, 128:129], 1e-30)
        g = s[:, :din] / den
        o = jnp.dot(g, w_ref[...], preferred_element_type=f32) + b_ref[...]
        o_ref[...] = jnp.maximum(o, 0.0) if relu else o

    return pl.pallas_call(
        body,
        grid=(NBLK,),
        in_specs=[pl.BlockSpec((2, BR, 144), lambda i: (0, i, 0)),
                  _full((din, dout)), _full((1, dout))],
        out_specs=_rows((BR, dout)),
        out_shape=jax.ShapeDtypeStruct((N1, dout), f32),
    )(accp, W, b)


def _finish_add(accps, b, widths, dout, relu, logsm):
    def body(*refs):
        a_refs = refs[:len(widths)]
        b_ref, o_ref = refs[-2], refs[-1]
        s0 = a_refs[0][0] + a_refs[0][1]
        den = jnp.maximum(s0[:, widths[0] - 16:widths[0] - 15], 1e-30)
        pieces = [s0[:, :widths[0] - 16]]
        for k in range(1, len(widths)):
            pieces.append(a_refs[k][0] + a_refs[k][1])
        g = jnp.concatenate(pieces, axis=1) if len(pieces) > 1 else pieces[0]
        t = g / den + b_ref[...]
        if relu:
            t = jnp.maximum(t, 0.0)
        if logsm:
            m = jnp.max(t, axis=1, keepdims=True)
            t = t - (m + jnp.log(jnp.sum(jnp.exp(t - m), axis=1,
                                         keepdims=True)))
        o_ref[...] = t

    return pl.pallas_call(
        body,
        grid=(NBLK,),
        in_specs=[pl.BlockSpec((2, BR, w), lambda i: (0, i, 0))
                  for w in widths] + [_full((1, dout))],
        out_specs=_rows((BR, dout)),
        out_shape=jax.ShapeDtypeStruct((N1, dout), f32),
    )(*accps, b)


@functools.lru_cache(maxsize=None)
def _make_sc_edge_pass(d_pad):
    mesh = plsc.VectorSubcoreMesh(core_axis_name="c", subcore_axis_name="s")

    @functools.partial(
        pl.kernel,
        out_type=jax.ShapeDtypeStruct((2, N1, d_pad), f32),
        mesh=mesh,
        compiler_params=pltpu.CompilerParams(needs_layout_passes=False,
                                             use_tc_tiling_on_sc=False),
        scratch_types=[
            pltpu.VMEM_SHARED((N1,), f32),
            pltpu.VMEM_SHARED((N1,), f32),
            pltpu.VMEM_SHARED((N1, d_pad), f32),
            [pltpu.VMEM((CH,), i32) for _ in range(2)],
            [pltpu.VMEM((CH,), i32) for _ in range(2)],
            [pltpu.VMEM((CH,), f32) for _ in range(2)],
            [pltpu.VMEM((CH,), f32) for _ in range(2)],
            [pltpu.VMEM((CH,), f32) for _ in range(2)],
            [pltpu.VMEM((CH, d_pad), f32) for _ in range(2)],
            pltpu.VMEM((16, d_pad), f32),
            pltpu.VMEM((16,), f32),
            [pltpu.SemaphoreType.DMA for _ in range(10)],
        ],
    )
    def sc_pass(m_hbm, src_hbm, dst_hbm, uv_hbm, g_hbm, out_hbm,
                u_sh, v_sh, acc, sv, dv, ub, vb, pv, rows, zbuf, gbuf,
                sems):
        cid = lax.axis_index("c")
        sid = lax.axis_index("s")
        nv = d_pad // 16
        ssv, sdv, sub_s, svb, srw = (sems[0:2], sems[2:4], sems[4:6],
                                     sems[6:8], sems[8:10])

        @pl.when(sid == 0)
        def _():
            pltpu.sync_copy(uv_hbm.at[0], u_sh)
            pltpu.sync_copy(uv_hbm.at[1], v_sh)

        def zrow(e, carry):
            for k in range(nv):
                zbuf[e, pl.ds(16 * k, 16)] = jnp.zeros((16,), f32)
            return carry
        lax.fori_loop(0, 16, zrow, 0)
        row0 = sid * RPS

        def zacc(k, carry):
            pltpu.sync_copy(zbuf, acc.at[pl.ds(row0 + 16 * k, 16)])
            return carry
        lax.fori_loop(0, RPS // 16, zacc, 0)

        pltpu.sync_copy(g_hbm.at[0], gbuf)
        gmax = jnp.max(gbuf[...])

        plsc.subcore_barrier()

        ebase = (cid * 16 + sid) * EPW

        def w1_issue(ch, s):
            b = ebase + ch * CH
            pltpu.async_copy(src_hbm.at[pl.ds(b, CH)], sv[s], ssv[s])
            pltpu.async_copy(dst_hbm.at[pl.ds(b, CH)], dv[s], sdv[s])

        def w1_wait(ch, s):
            b = ebase + ch * CH
            pltpu.make_async_copy(src_hbm.at[pl.ds(b, CH)], sv[s],
                                  ssv[s]).wait()
            pltpu.make_async_copy(dst_hbm.at[pl.ds(b, CH)], dv[s],
                                  sdv[s]).wait()

        def w2_issue(s):
            pltpu.async_copy(u_sh.at[sv[s]], ub[s], sub_s[s])
            pltpu.async_copy(v_sh.at[dv[s]], vb[s], svb[s])
            pltpu.async_copy(m_hbm.at[sv[s]], rows[s], srw[s])

        def w2_wait(s):
            pltpu.make_async_copy(u_sh.at[sv[s]], ub[s], sub_s[s]).wait()
            pltpu.make_async_copy(v_sh.at[dv[s]], vb[s], svb[s]).wait()
            pltpu.make_async_copy(m_hbm.at[sv[s]], rows[s], srw[s]).wait()

        w1_issue(0, 0)
        w1_wait(0, 0)
        w2_issue(0)
        w1_issue(1, 1)

        def pair_body(g, carry):
            for s in range(2):
                ch = 2 * g + s
                o = 1 - s
                w2_wait(s)

                @plsc.parallel_loop(0, CH // 16)
                def _(j):
                    ug = ub[s][pl.ds(16 * j, 16)]
                    vg = vb[s][pl.ds(16 * j, 16)]
                    zz = ug + vg
                    e = jnp.maximum(zz, 0.2 * zz)
                    zub = gmax + vg
                    cc = jnp.maximum(zub, 0.2 * zub)
                    pv[s][pl.ds(16 * j, 16)] = jnp.exp(e - cc)

                @plsc.parallel_loop(0, CH, unroll=2)
                def _(e_i):
                    pb = plsc.load_gather(pv[s], [jnp.full((16,), e_i, i32)])
                    for k in range(nv):
                        sl = pl.ds(16 * k, 16)
                        rows[s][e_i, sl] = rows[s][e_i, sl] * pb

                pltpu.sync_copy(rows[s], acc.at[dv[s]], add=True)

                @pl.when(ch + 2 < NCH)
                def _():
                    w1_issue(ch + 2, s)

                @pl.when(ch + 1 < NCH)
                def _():
                    w1_wait(ch + 1, o)
                    w2_issue(o)
            return carry
        lax.fori_loop(0, NCH // 2, pair_body, 0)

        plsc.subcore_barrier()
        for k in range(RPS // 128):
            r0 = row0 + k * 128
            pltpu.sync_copy(acc.at[pl.ds(r0, 128)],
                            out_hbm.at[cid, pl.ds(r0, 128)])

    return sc_pass


def kernel(x, edge_index, W1, a1_src, a1_dst, b1, W2, a2_src, a2_dst, b2,
           W3, a3_src, a3_dst, b3, W4, a4_src, a4_dst, b4):
    loops = jnp.arange(N, dtype=edge_index.dtype)
    pad = jnp.full((E1 - E_REAL,), N, i32)
    src = jnp.concatenate([edge_index[0], loops, pad])
    dst = jnp.concatenate([edge_index[1], loops, pad])

    x0 = jnp.zeros((N1, 128), f32).at[:N].set(x.astype(f32))

    def col(a):
        return a.astype(f32).reshape(-1, 1)

    def row(b):
        return b.astype(f32).reshape(1, -1)

    sc128 = _make_sc_edge_pass(144)
    sc_l4 = _make_sc_edge_pass(32)

    m, uv, g = _prep_pre(x0, W1, col(a1_src), col(a1_dst))
    accp = sc128(m, src, dst, uv, g)
    h = _finish_matmul(accp, W1, row(b1), relu=True)

    m, uv, g = _prep_pre(h, W2, col(a2_src), col(a2_dst))
    accp = sc128(m, src, dst, uv, g)
    h = _finish_matmul(accp, W2, row(b2), relu=True)

    widths3 = (144, 128, 128, 128)
    outs = _prep_post(h, W3, col(a3_src), col(a3_dst), widths3)
    ms, uv, g = outs[:-2], outs[-2], outs[-1]
    accps = [_make_sc_edge_pass(w)(mk, src, dst, uv, g) for w, mk in
             zip(widths3, ms)]
    h = _finish_add(accps, row(b3), widths3, 512, relu=True, logsm=False)

    outs = _prep_post(h, W4, col(a4_src), col(a4_dst), (32,))
    m, uv, g = outs[0], outs[1], outs[2]
    accp = sc_l4(m, src, dst, uv, g)
    out = _finish_add([accp], row(b4), (32,), 16, relu=False, logsm=True)

    return out[:N]
